# Initial kernel scaffold; baseline (speedup 1.0000x reference)
import jax
import jax.numpy as jnp
from jax.experimental import pallas as pl


def kernel(x, W1, W2, W3, W4, W5):
    raise NotImplementedError("write your pallas kernel here")


# speedup vs baseline: 7.7825x; 7.7825x over previous
import functools

import jax
import jax.numpy as jnp
from jax import lax
from jax.experimental import pallas as pl
from jax.experimental.pallas import tpu as pltpu
from jax.experimental.pallas import tpu_sc as plsc

KNN = 30
KPAD = 32
ROWS = 256
CROWS = 64
NWORKERS = 32
GRP = 4


def _knn_tc_body(xT_ref, x_ref, wbT_ref, idx_ref, yb_ref, *, n):
    b = pl.program_id(0)
    xrows = xT_ref[0]
    xfull = x_ref[0]

    dot = lax.dot_general(
        xrows, xfull, (((1,), (0,)), ((), ())),
        preferred_element_type=jnp.float32)
    nm = jnp.sum(xfull * xfull, axis=0, keepdims=True)
    nr = jnp.sum(xrows * xrows, axis=1, keepdims=True)
    d = 2.0 * dot - nm - nr

    iota = lax.broadcasted_iota(jnp.int32, d.shape, 1)
    neg_inf = jnp.float32(-jnp.inf)
    cols = []
    for _ in range(KNN):
        v = jnp.max(d, axis=1, keepdims=True)
        cand = jnp.where(d == v, iota, n)
        am = jnp.min(cand, axis=1, keepdims=True)
        cols.append(am)
        d = jnp.where(iota == am, neg_inf, d)
    cols += [cols[-1]] * (KPAD - KNN)
    idx_ref[0] = jnp.concatenate(cols, axis=1) + b * n

    yb_ref[0] = jnp.dot(xrows, wbT_ref[...], preferred_element_type=jnp.float32)


def _knn_tc(xT, x, wbT):
    bsz, n, twin = xT.shape
    two = wbT.shape[1]
    grid = (bsz, n // ROWS)
    return pl.pallas_call(
        functools.partial(_knn_tc_body, n=n),
        grid=grid,
        in_specs=[
            pl.BlockSpec((1, ROWS, twin), lambda b, i: (b, i, 0)),
            pl.BlockSpec((1, twin, n), lambda b, i: (b, 0, 0)),
            pl.BlockSpec((twin, two), lambda b, i: (0, 0)),
        ],
        out_specs=(
            pl.BlockSpec((1, ROWS, KPAD), lambda b, i: (b, i, 0)),
            pl.BlockSpec((1, ROWS, two), lambda b, i: (b, i, 0)),
        ),
        out_shape=(
            jax.ShapeDtypeStruct((bsz, n, KPAD), jnp.int32),
            jax.ShapeDtypeStruct((bsz, n, two), jnp.float32),
        ),
    )(xT, x, wbT)


def _gather_sc(total, twin, cp):
    ppw = total // NWORKERS
    ngroups = ppw // GRP
    nrows = GRP * KPAD
    mesh = plsc.VectorSubcoreMesh(core_axis_name="c", subcore_axis_name="s")

    @functools.partial(
        pl.kernel, mesh=mesh,
        out_type=jax.ShapeDtypeStruct((total * KPAD * cp,), jnp.float32),
        scratch_types=[
            pltpu.VMEM((ppw * KPAD,), jnp.int32),
            pltpu.VMEM((nrows, twin), jnp.float32),
            pltpu.VMEM((GRP, twin), jnp.float32),
            pltpu.VMEM((nrows * cp,), jnp.float32),
            pltpu.SemaphoreType.DMA,
        ],
    )
    def k(tab_hbm, idxf_hbm, outf_hbm, idx_v, rows_v, ctr_v, pack_v, sem):
        wid = lax.axis_index("s") * 2 + lax.axis_index("c")
        base = wid * ppw
        pltpu.sync_copy(idxf_hbm.at[pl.ds(base * KPAD, ppw * KPAD)], idx_v)

        def grp_body(gi, carry):
            p0 = base + gi * GRP
            pltpu.async_copy(
                tab_hbm.at[idx_v.at[pl.ds(gi * nrows, nrows)]], rows_v, sem
            ).wait()
            pltpu.sync_copy(tab_hbm.at[pl.ds(p0, GRP)], ctr_v)
            for p in range(GRP):
                def cbody(ci, c2, _p=p):
                    s = ci * 16
                    ctr = ctr_v[_p, pl.ds(s, 16)]
                    for j in range(KPAD):
                        r = _p * KPAD + j
                        pack_v[pl.ds(r * cp + s, 16)] = (
                            rows_v[r, pl.ds(s, 16)] - ctr)
                    return c2
                lax.fori_loop(0, cp // 16, cbody, 0)
            pltpu.sync_copy(pack_v, outf_hbm.at[pl.ds(p0 * KPAD * cp,
                                                      nrows * cp)])
            return carry

        lax.fori_loop(0, ngroups, grp_body, 0)

    return k


def _conv_tc_body(diff_ref, waT_ref, yb_ref, out_ref):
    e = jnp.dot(diff_ref[...], waT_ref[...],
                preferred_element_type=jnp.float32)
    m = jnp.max(e.reshape(CROWS, KPAD, e.shape[1]), axis=1)
    h = m + yb_ref[...]
    out_ref[...] = jnp.maximum(h, 0.2 * h)


def _conv_tc(diff, waT, yb):
    rows, cp = diff.shape
    two = waT.shape[1]
    total = rows // KPAD
    grid = (total // CROWS,)
    return pl.pallas_call(
        _conv_tc_body,
        grid=grid,
        in_specs=[
            pl.BlockSpec((CROWS * KPAD, cp), lambda i: (i, 0)),
            pl.BlockSpec((cp, two), lambda i: (0, 0)),
            pl.BlockSpec((CROWS, two), lambda i: (i, 0)),
        ],
        out_specs=pl.BlockSpec((CROWS, two), lambda i: (i, 0)),
        out_shape=jax.ShapeDtypeStruct((total, two), jnp.float32),
    )(diff, waT, yb)


def _final_body(x1_ref, x2_ref, x3_ref, x4_ref, w1_ref, w2_ref, w3_ref,
                w4_ref, out_ref):
    h = jnp.dot(x1_ref[0], w1_ref[...], preferred_element_type=jnp.float32)
    h += jnp.dot(x2_ref[0], w2_ref[...], preferred_element_type=jnp.float32)
    h += jnp.dot(x3_ref[0], w3_ref[...], preferred_element_type=jnp.float32)
    h += jnp.dot(x4_ref[0], w4_ref[...], preferred_element_type=jnp.float32)
    h = jnp.maximum(h, 0.2 * h)
    m = jnp.max(h, axis=0, keepdims=True)

    @pl.when(pl.program_id(1) == 0)
    def _():
        out_ref[0] = m

    @pl.when(pl.program_id(1) != 0)
    def _():
        out_ref[0] = jnp.maximum(out_ref[0], m)


def _final_tc(x1, x2, x3, x4, w51, w52, w53, w54):
    bsz, n, _ = x1.shape
    oo = w51.shape[1]
    grid = (bsz, n // ROWS)
    ispec = lambda arr: pl.BlockSpec((1, ROWS, arr.shape[2]),
                                     lambda b, i: (b, i, 0))
    wspec = lambda w: pl.BlockSpec((w.shape[0], oo), lambda b, i: (0, 0))
    return pl.pallas_call(
        _final_body,
        grid=grid,
        in_specs=[ispec(x1), ispec(x2), ispec(x3), ispec(x4),
                  wspec(w51), wspec(w52), wspec(w53), wspec(w54)],
        out_specs=pl.BlockSpec((1, 1, oo), lambda b, i: (b, 0, 0)),
        out_shape=jax.ShapeDtypeStruct((bsz, 1, oo), jnp.float32),
    )(x1, x2, x3, x4, w51, w52, w53, w54).reshape(bsz, oo)


def _pad16(v):
    return max(16, -(-v // 16) * 16)


def kernel(x, W1, W2, W3, W4, W5):
    bsz, c0, n = x.shape
    total = bsz * n
    twin = 128

    xT = jnp.transpose(x, (0, 2, 1)).reshape(total, c0)
    xT = jnp.pad(xT, ((0, 0), (0, twin - c0)))
    cur_c = c0

    feats = []
    for w in (W1, W2, W3, W4):
        o, c2 = w.shape
        c = c2 // 2
        cp = _pad16(cur_c)
        two = max(o, 128)
        waT = jnp.pad(w[:, :c].T, ((0, cp - c), (0, two - o)))
        wbT = jnp.pad(w[:, c:].T, ((0, twin - c), (0, two - o)))

        xT3 = xT.reshape(bsz, n, twin)
        xfull = jnp.transpose(xT3, (0, 2, 1))
        idx, yb = _knn_tc(xT3, xfull, wbT)
        diff = _gather_sc(total, twin, cp)(
            xT, idx.reshape(total * KPAD)).reshape(total * KPAD, cp)
        nxt = _conv_tc(diff, waT, yb.reshape(total, two))
        feats.append(nxt.reshape(bsz, n, two))
        xT = nxt
        cur_c = o

    sizes = (W1.shape[0], W2.shape[0], W3.shape[0], W4.shape[0])
    offs = [0]
    for s in sizes:
        offs.append(offs[-1] + s)
    wparts = []
    for i, f in enumerate(feats):
        wpart = W5[:, offs[i]:offs[i + 1]].T
        wparts.append(jnp.pad(wpart, ((0, f.shape[2] - sizes[i]), (0, 0))))
    return _final_tc(feats[0], feats[1], feats[2], feats[3], *wparts)


# speedup vs baseline: 9.1752x; 1.1790x over previous
import functools

import jax
import jax.numpy as jnp
from jax import lax
from jax.experimental import pallas as pl
from jax.experimental.pallas import tpu as pltpu
from jax.experimental.pallas import tpu_sc as plsc

KNN = 30
KPAD = 32
ROWS = 256
CROWS = 64
NWORKERS = 32
GRP = 4


def _knn_tc_body(xT_ref, x_ref, wbT_ref, idx_ref, yb_ref, *, n):
    b = pl.program_id(0)
    xrows = xT_ref[0]
    xfull = x_ref[0]

    dot = lax.dot_general(
        xrows, xfull, (((1,), (0,)), ((), ())),
        preferred_element_type=jnp.float32)
    nm = jnp.sum(xfull * xfull, axis=0, keepdims=True)
    nr = jnp.sum(xrows * xrows, axis=1, keepdims=True)
    d = 2.0 * dot - nm - nr

    iota = lax.broadcasted_iota(jnp.int32, d.shape, 1)
    neg_inf = jnp.float32(-jnp.inf)
    cols = []
    for _ in range(KNN):
        am = jnp.argmax(d, axis=1)[:, None]
        cols.append(am)
        d = jnp.where(iota == am, neg_inf, d)
    cols += [cols[-1]] * (KPAD - KNN)
    idx_ref[0] = jnp.concatenate(cols, axis=1) + b * n

    yb_ref[0] = jnp.dot(xrows, wbT_ref[...], preferred_element_type=jnp.float32)


def _knn_tc(xT, x, wbT):
    bsz, n, twin = xT.shape
    two = wbT.shape[1]
    grid = (bsz, n // ROWS)
    return pl.pallas_call(
        functools.partial(_knn_tc_body, n=n),
        grid=grid,
        in_specs=[
            pl.BlockSpec((1, ROWS, twin), lambda b, i: (b, i, 0)),
            pl.BlockSpec((1, twin, n), lambda b, i: (b, 0, 0)),
            pl.BlockSpec((twin, two), lambda b, i: (0, 0)),
        ],
        out_specs=(
            pl.BlockSpec((1, ROWS, KPAD), lambda b, i: (b, i, 0)),
            pl.BlockSpec((1, ROWS, two), lambda b, i: (b, i, 0)),
        ),
        out_shape=(
            jax.ShapeDtypeStruct((bsz, n, KPAD), jnp.int32),
            jax.ShapeDtypeStruct((bsz, n, two), jnp.float32),
        ),
    )(xT, x, wbT)


def _gather_sc(total, twin, cp):
    ppw = total // NWORKERS
    ngroups = ppw // GRP
    nrows = GRP * KPAD
    mesh = plsc.VectorSubcoreMesh(core_axis_name="c", subcore_axis_name="s")

    @functools.partial(
        pl.kernel, mesh=mesh,
        out_type=jax.ShapeDtypeStruct((total * KPAD * cp,), jnp.float32),
        scratch_types=[
            pltpu.VMEM((ppw * KPAD,), jnp.int32),
            pltpu.VMEM((nrows, twin), jnp.float32),
            pltpu.VMEM((GRP, twin), jnp.float32),
            pltpu.VMEM((nrows * cp,), jnp.float32),
            pltpu.SemaphoreType.DMA,
        ],
    )
    def k(tab_hbm, idxf_hbm, outf_hbm, idx_v, rows_v, ctr_v, pack_v, sem):
        wid = lax.axis_index("s") * 2 + lax.axis_index("c")
        base = wid * ppw
        pltpu.sync_copy(idxf_hbm.at[pl.ds(base * KPAD, ppw * KPAD)], idx_v)

        def grp_body(gi, carry):
            p0 = base + gi * GRP
            pltpu.async_copy(
                tab_hbm.at[idx_v.at[pl.ds(gi * nrows, nrows)]], rows_v, sem
            ).wait()
            pltpu.sync_copy(tab_hbm.at[pl.ds(p0, GRP)], ctr_v)
            for p in range(GRP):
                def cbody(ci, c2, _p=p):
                    s = ci * 16
                    ctr = ctr_v[_p, pl.ds(s, 16)]
                    for j in range(KPAD):
                        r = _p * KPAD + j
                        pack_v[pl.ds(r * cp + s, 16)] = (
                            rows_v[r, pl.ds(s, 16)] - ctr)
                    return c2
                lax.fori_loop(0, cp // 16, cbody, 0)
            pltpu.sync_copy(pack_v, outf_hbm.at[pl.ds(p0 * KPAD * cp,
                                                      nrows * cp)])
            return carry

        lax.fori_loop(0, ngroups, grp_body, 0)

    return k


def _conv_tc_body(diff_ref, waT_ref, yb_ref, out_ref):
    e = jnp.dot(diff_ref[...], waT_ref[...],
                preferred_element_type=jnp.float32)
    m = jnp.max(e.reshape(CROWS, KPAD, e.shape[1]), axis=1)
    h = m + yb_ref[...]
    out_ref[...] = jnp.maximum(h, 0.2 * h)


def _conv_tc(diff, waT, yb):
    rows, cp = diff.shape
    two = waT.shape[1]
    total = rows // KPAD
    grid = (total // CROWS,)
    return pl.pallas_call(
        _conv_tc_body,
        grid=grid,
        in_specs=[
            pl.BlockSpec((CROWS * KPAD, cp), lambda i: (i, 0)),
            pl.BlockSpec((cp, two), lambda i: (0, 0)),
            pl.BlockSpec((CROWS, two), lambda i: (i, 0)),
        ],
        out_specs=pl.BlockSpec((CROWS, two), lambda i: (i, 0)),
        out_shape=jax.ShapeDtypeStruct((total, two), jnp.float32),
    )(diff, waT, yb)


def _final_body(x1_ref, x2_ref, x3_ref, x4_ref, w1_ref, w2_ref, w3_ref,
                w4_ref, out_ref):
    h = jnp.dot(x1_ref[0], w1_ref[...], preferred_element_type=jnp.float32)
    h += jnp.dot(x2_ref[0], w2_ref[...], preferred_element_type=jnp.float32)
    h += jnp.dot(x3_ref[0], w3_ref[...], preferred_element_type=jnp.float32)
    h += jnp.dot(x4_ref[0], w4_ref[...], preferred_element_type=jnp.float32)
    h = jnp.maximum(h, 0.2 * h)
    m = jnp.max(h, axis=0, keepdims=True)

    @pl.when(pl.program_id(1) == 0)
    def _():
        out_ref[0] = m

    @pl.when(pl.program_id(1) != 0)
    def _():
        out_ref[0] = jnp.maximum(out_ref[0], m)


def _final_tc(x1, x2, x3, x4, w51, w52, w53, w54):
    bsz, n, _ = x1.shape
    oo = w51.shape[1]
    grid = (bsz, n // ROWS)
    ispec = lambda arr: pl.BlockSpec((1, ROWS, arr.shape[2]),
                                     lambda b, i: (b, i, 0))
    wspec = lambda w: pl.BlockSpec((w.shape[0], oo), lambda b, i: (0, 0))
    return pl.pallas_call(
        _final_body,
        grid=grid,
        in_specs=[ispec(x1), ispec(x2), ispec(x3), ispec(x4),
                  wspec(w51), wspec(w52), wspec(w53), wspec(w54)],
        out_specs=pl.BlockSpec((1, 1, oo), lambda b, i: (b, 0, 0)),
        out_shape=jax.ShapeDtypeStruct((bsz, 1, oo), jnp.float32),
    )(x1, x2, x3, x4, w51, w52, w53, w54).reshape(bsz, oo)


def _pad16(v):
    return max(16, -(-v // 16) * 16)


def kernel(x, W1, W2, W3, W4, W5):
    bsz, c0, n = x.shape
    total = bsz * n
    twin = 128

    xT = jnp.transpose(x, (0, 2, 1)).reshape(total, c0)
    xT = jnp.pad(xT, ((0, 0), (0, twin - c0)))
    cur_c = c0

    feats = []
    for w in (W1, W2, W3, W4):
        o, c2 = w.shape
        c = c2 // 2
        cp = _pad16(cur_c)
        two = max(o, 128)
        waT = jnp.pad(w[:, :c].T, ((0, cp - c), (0, two - o)))
        wbT = jnp.pad(w[:, c:].T, ((0, twin - c), (0, two - o)))

        xT3 = xT.reshape(bsz, n, twin)
        xfull = jnp.transpose(xT3, (0, 2, 1))
        idx, yb = _knn_tc(xT3, xfull, wbT)
        diff = _gather_sc(total, twin, cp)(
            xT, idx.reshape(total * KPAD)).reshape(total * KPAD, cp)
        nxt = _conv_tc(diff, waT, yb.reshape(total, two))
        feats.append(nxt.reshape(bsz, n, two))
        xT = nxt
        cur_c = o

    sizes = (W1.shape[0], W2.shape[0], W3.shape[0], W4.shape[0])
    offs = [0]
    for s in sizes:
        offs.append(offs[-1] + s)
    wparts = []
    for i, f in enumerate(feats):
        wpart = W5[:, offs[i]:offs[i + 1]].T
        wparts.append(jnp.pad(wpart, ((0, f.shape[2] - sizes[i]), (0, 0))))
    return _final_tc(feats[0], feats[1], feats[2], feats[3], *wparts)


# speedup vs baseline: 10.5331x; 1.1480x over previous
import functools

import jax
import jax.numpy as jnp
from jax import lax
from jax.experimental import pallas as pl
from jax.experimental.pallas import tpu as pltpu
from jax.experimental.pallas import tpu_sc as plsc

KNN = 30
KPAD = 32
ROWS = 256
CROWS = 64
NWORKERS = 32
GRP = 4


def _knn_tc_body(xT_ref, x_ref, wbT_ref, idx_ref, yb_ref, *, n):
    b = pl.program_id(0)
    xrows = xT_ref[0]
    xfull = x_ref[0]

    dot = lax.dot_general(
        xrows, xfull, (((1,), (0,)), ((), ())),
        preferred_element_type=jnp.float32)
    nm = jnp.sum(xfull * xfull, axis=0, keepdims=True)
    nr = jnp.sum(xrows * xrows, axis=1, keepdims=True)
    d = 2.0 * dot - nm - nr

    iota = lax.broadcasted_iota(jnp.int32, d.shape, 1)
    neg_inf = jnp.float32(-jnp.inf)
    cols = []
    for _ in range(KNN):
        am = jnp.argmax(d, axis=1)[:, None]
        cols.append(am)
        d = jnp.where(iota == am, neg_inf, d)
    cols += [cols[-1]] * (KPAD - KNN)
    idx_ref[0] = jnp.concatenate(cols, axis=1) + b * n

    yb_ref[0] = jnp.dot(xrows, wbT_ref[...], preferred_element_type=jnp.float32)


def _knn_tc(xT, x, wbT):
    bsz, n, twin = xT.shape
    two = wbT.shape[1]
    grid = (bsz, n // ROWS)
    return pl.pallas_call(
        functools.partial(_knn_tc_body, n=n),
        grid=grid,
        in_specs=[
            pl.BlockSpec((1, ROWS, twin), lambda b, i: (b, i, 0)),
            pl.BlockSpec((1, twin, n), lambda b, i: (b, 0, 0)),
            pl.BlockSpec((twin, two), lambda b, i: (0, 0)),
        ],
        out_specs=(
            pl.BlockSpec((1, ROWS, KPAD), lambda b, i: (b, i, 0)),
            pl.BlockSpec((1, ROWS, two), lambda b, i: (b, i, 0)),
        ),
        out_shape=(
            jax.ShapeDtypeStruct((bsz, n, KPAD), jnp.int32),
            jax.ShapeDtypeStruct((bsz, n, two), jnp.float32),
        ),
    )(xT, x, wbT)


def _gather_sc(total, twin, cp):
    ppw = total // NWORKERS
    ngroups = ppw // GRP
    nrows = GRP * KPAD
    mesh = plsc.VectorSubcoreMesh(core_axis_name="c", subcore_axis_name="s")

    @functools.partial(
        pl.kernel, mesh=mesh,
        out_type=jax.ShapeDtypeStruct((total * KPAD * cp,), jnp.float32),
        scratch_types=[
            pltpu.VMEM((ppw * KPAD,), jnp.int32),
            pltpu.VMEM((nrows, twin), jnp.float32),
            pltpu.VMEM((GRP, twin), jnp.float32),
            pltpu.VMEM((nrows * cp,), jnp.float32),
            pltpu.SemaphoreType.DMA,
        ],
    )
    def k(tab_hbm, idxf_hbm, outf_hbm, idx_v, rows_v, ctr_v, pack_v, sem):
        wid = lax.axis_index("s") * 2 + lax.axis_index("c")
        base = wid * ppw
        pltpu.sync_copy(idxf_hbm.at[pl.ds(base * KPAD, ppw * KPAD)], idx_v)

        def grp_body(gi, carry):
            p0 = base + gi * GRP
            pltpu.async_copy(
                tab_hbm.at[idx_v.at[pl.ds(gi * nrows, nrows)]], rows_v, sem
            ).wait()
            pltpu.sync_copy(tab_hbm.at[pl.ds(p0, GRP)], ctr_v)
            for p in range(GRP):
                def cbody(ci, c2, _p=p):
                    s = ci * 16
                    ctr = ctr_v[_p, pl.ds(s, 16)]
                    for j in range(KPAD):
                        r = _p * KPAD + j
                        pack_v[pl.ds(r * cp + s, 16)] = (
                            rows_v[r, pl.ds(s, 16)] - ctr)
                    return c2
                lax.fori_loop(0, cp // 16, cbody, 0)
            pltpu.sync_copy(pack_v, outf_hbm.at[pl.ds(p0 * KPAD * cp,
                                                      nrows * cp)])
            return carry

        lax.fori_loop(0, ngroups, grp_body, 0)

    return k


def _conv_tc_body(diff_ref, waT_ref, yb_ref, out_ref):
    e = jnp.dot(diff_ref[...], waT_ref[...],
                preferred_element_type=jnp.float32)
    m = jnp.max(e.reshape(CROWS, KPAD, e.shape[1]), axis=1)
    h = m + yb_ref[...]
    out_ref[...] = jnp.maximum(h, 0.2 * h)


def _conv_tc(diff, waT, yb):
    rows, cp = diff.shape
    two = waT.shape[1]
    total = rows // KPAD
    grid = (total // CROWS,)
    return pl.pallas_call(
        _conv_tc_body,
        grid=grid,
        in_specs=[
            pl.BlockSpec((CROWS * KPAD, cp), lambda i: (i, 0)),
            pl.BlockSpec((cp, two), lambda i: (0, 0)),
            pl.BlockSpec((CROWS, two), lambda i: (i, 0)),
        ],
        out_specs=pl.BlockSpec((CROWS, two), lambda i: (i, 0)),
        out_shape=jax.ShapeDtypeStruct((total, two), jnp.float32),
    )(diff, waT, yb)


def _final_body(x1_ref, x2_ref, x3_ref, x4_ref, w1_ref, w2_ref, w3_ref,
                w4_ref, out_ref):
    h = jnp.dot(x1_ref[0], w1_ref[...], preferred_element_type=jnp.float32)
    h += jnp.dot(x2_ref[0], w2_ref[...], preferred_element_type=jnp.float32)
    h += jnp.dot(x3_ref[0], w3_ref[...], preferred_element_type=jnp.float32)
    h += jnp.dot(x4_ref[0], w4_ref[...], preferred_element_type=jnp.float32)
    h = jnp.maximum(h, 0.2 * h)
    m = jnp.max(h, axis=0, keepdims=True)

    @pl.when(pl.program_id(1) == 0)
    def _():
        out_ref[0] = m

    @pl.when(pl.program_id(1) != 0)
    def _():
        out_ref[0] = jnp.maximum(out_ref[0], m)


def _final_tc(x1, x2, x3, x4, w51, w52, w53, w54):
    bsz, n, _ = x1.shape
    oo = w51.shape[1]
    grid = (bsz, n // ROWS)
    ispec = lambda arr: pl.BlockSpec((1, ROWS, arr.shape[2]),
                                     lambda b, i: (b, i, 0))
    wspec = lambda w: pl.BlockSpec((w.shape[0], oo), lambda b, i: (0, 0))
    return pl.pallas_call(
        _final_body,
        grid=grid,
        in_specs=[ispec(x1), ispec(x2), ispec(x3), ispec(x4),
                  wspec(w51), wspec(w52), wspec(w53), wspec(w54)],
        out_specs=pl.BlockSpec((1, 1, oo), lambda b, i: (b, 0, 0)),
        out_shape=jax.ShapeDtypeStruct((bsz, 1, oo), jnp.float32),
    )(x1, x2, x3, x4, w51, w52, w53, w54).reshape(bsz, oo)


def _pad16(v):
    return max(16, -(-v // 16) * 16)


def kernel(x, W1, W2, W3, W4, W5):
    bsz, c0, n = x.shape
    twin = 128

    ws = []
    cur_c = c0
    for w in (W1, W2, W3, W4):
        o, c2 = w.shape
        c = c2 // 2
        cp = _pad16(cur_c)
        two = max(o, 128)
        waT = jnp.pad(w[:, :c].T, ((0, cp - c), (0, two - o)))
        wbT = jnp.pad(w[:, c:].T, ((0, twin - c), (0, two - o)))
        ws.append((waT, wbT, cp, two))
        cur_c = o

    sizes = (W1.shape[0], W2.shape[0], W3.shape[0], W4.shape[0])
    offs = [0]
    for s in sizes:
        offs.append(offs[-1] + s)

    xT0 = jnp.transpose(x, (0, 2, 1))
    xT0 = jnp.pad(xT0, ((0, 0), (0, 0), (0, twin - c0)))

    outs = []
    wparts = None
    for b in range(bsz):
        xT = xT0[b]
        feats = []
        for (waT, wbT, cp, two) in ws:
            xT3 = xT.reshape(1, n, xT.shape[1])
            xfull = jnp.transpose(xT3, (0, 2, 1))
            idx, yb = _knn_tc(xT3, xfull, wbT)
            diff = _gather_sc(n, twin, cp)(
                xT, idx.reshape(n * KPAD)).reshape(n * KPAD, cp)
            nxt = _conv_tc(diff, waT, yb.reshape(n, two))
            feats.append(nxt.reshape(1, n, two))
            xT = nxt
        if wparts is None:
            wparts = []
            for i, f in enumerate(feats):
                wpart = W5[:, offs[i]:offs[i + 1]].T
                wparts.append(
                    jnp.pad(wpart, ((0, f.shape[2] - sizes[i]), (0, 0))))
        outs.append(_final_tc(feats[0], feats[1], feats[2], feats[3], *wparts))
    return jnp.concatenate(outs, axis=0)
